# baseline (speedup 1.0000x reference)
import functools

import jax
import jax.numpy as jnp
from jax import lax
from jax.experimental import pallas as pl
from jax.experimental.pallas import tpu as pltpu

B = 4096
E = 8
D = 1024
H = 2048
O = 1024
CAP = B // E
BLK = 512


def _k1_body(x_ref, wb_ref, bb_ref, wg_ref, bg_ref, feat_ref, sc_ref, gp_ref):
    f = jnp.dot(x_ref[...], wb_ref[...], preferred_element_type=jnp.float32)
    f = jnp.maximum(f + bb_ref[...], 0.0)
    feat_ref[...] = f
    s = jnp.dot(f, wg_ref[...], preferred_element_type=jnp.float32) + bg_ref[...]
    sc_ref[...] = s
    m = jnp.max(s, axis=1, keepdims=True)
    ex = jnp.exp(s - m)
    gp_ref[...] = ex / jnp.sum(ex, axis=1, keepdims=True)


@jax.jit
def _k1(x, Wb, bb, Wg, bg):
    return pl.pallas_call(
        _k1_body,
        grid=(B // BLK,),
        in_specs=[
            pl.BlockSpec((BLK, D), lambda i: (i, 0)),
            pl.BlockSpec((D, D), lambda i: (0, 0)),
            pl.BlockSpec((1, D), lambda i: (0, 0)),
            pl.BlockSpec((D, E), lambda i: (0, 0)),
            pl.BlockSpec((1, E), lambda i: (0, 0)),
        ],
        out_specs=[
            pl.BlockSpec((BLK, D), lambda i: (i, 0)),
            pl.BlockSpec((BLK, E), lambda i: (i, 0)),
            pl.BlockSpec((BLK, E), lambda i: (i, 0)),
        ],
        out_shape=[
            jax.ShapeDtypeStruct((B, D), jnp.float32),
            jax.ShapeDtypeStruct((B, E), jnp.float32),
            jax.ShapeDtypeStruct((B, E), jnp.float32),
        ],
        compiler_params=pltpu.CompilerParams(
            dimension_semantics=("arbitrary",)),
    )(x, Wb, bb.reshape(1, D), Wg, bg.reshape(1, E))


def _k2_body(sorted_ref, order_ref, pos_ref, caps_ref, nfull_ref,
             assigned_ref):
    for e in range(E):
        caps_ref[e] = CAP
    nfull_ref[0] = 0
    nfull_ref[1] = 0

    def init_b(b, _):
        assigned_ref[b] = -1
        return 0
    lax.fori_loop(0, B, init_b, 0, unroll=8)

    def chunk(c, _):
        @pl.when(nfull_ref[0] < E - 1)
        def _():
            def step(i, _):
                idx = sorted_ref[c * 1024 + i]
                b = lax.shift_right_logical(idx, 3)
                e = lax.bitwise_and(idx, 7)
                cap = caps_ref[e]
                a = assigned_ref[b]
                take = jnp.logical_and(a < 0, cap > 0)
                ti = take.astype(jnp.int32)
                assigned_ref[b] = jnp.where(take, e, a)
                caps_ref[e] = cap - ti
                filled = ti * (cap == 1).astype(jnp.int32)
                nfull_ref[0] = nfull_ref[0] + filled
                nfull_ref[1] = nfull_ref[1] + filled * e
                return 0
            lax.fori_loop(0, 1024, step, 0)
        return 0
    lax.fori_loop(0, (B * E) // 1024, chunk, 0)

    e_last = (E * (E - 1)) // 2 - nfull_ref[1]

    for e in range(E):
        caps_ref[e] = 0

    def place(b, _):
        a = assigned_ref[b]
        e = jnp.where(a < 0, e_last, a)
        k = caps_ref[e]
        p = e * CAP + k
        order_ref[p] = b
        pos_ref[b] = p
        caps_ref[e] = k + 1
        return 0
    lax.fori_loop(0, B, place, 0)


@jax.jit
def _k2(sorted_idx):
    return pl.pallas_call(
        _k2_body,
        in_specs=[pl.BlockSpec(memory_space=pltpu.SMEM)],
        out_specs=[pl.BlockSpec(memory_space=pltpu.SMEM),
                   pl.BlockSpec(memory_space=pltpu.SMEM)],
        out_shape=[jax.ShapeDtypeStruct((B,), jnp.int32),
                   jax.ShapeDtypeStruct((B,), jnp.int32)],
        scratch_shapes=[
            pltpu.SMEM((E,), jnp.int32),
            pltpu.SMEM((2,), jnp.int32),
            pltpu.SMEM((B,), jnp.int32),
        ],
    )(sorted_idx)


HJ = 2
HB = H // HJ


def _k3_body(feat_ref, gp_ref, order_ref, w1_ref, b1_ref, w2_ref, b2_ref,
             o_ref, xs_ref, gs_ref, ya_ref):
    e = pl.program_id(0)
    j = pl.program_id(1)
    lane = lax.broadcasted_iota(jnp.int32, (1, E), 1)

    @pl.when(j == 0)
    def _():
        def gather_row(i, _):
            tok = order_ref[e * CAP + i]
            xs_ref[pl.ds(i, 1), :] = feat_ref[pl.ds(tok, 1), :]
            row = gp_ref[pl.ds(tok, 1), :]
            gs_ref[pl.ds(i, 1), :] = jnp.sum(
                jnp.where(lane == e, row, 0.0), axis=1, keepdims=True)
            return 0
        lax.fori_loop(0, CAP, gather_row, 0)

    h = jnp.dot(xs_ref[...], w1_ref[...], preferred_element_type=jnp.float32)
    h = jnp.maximum(h + b1_ref[0], 0.0)
    y = jnp.dot(h, w2_ref[...], preferred_element_type=jnp.float32)

    @pl.when(j == 0)
    def _():
        ya_ref[...] = y

    @pl.when(j > 0)
    def _():
        ya_ref[...] = ya_ref[...] + y

    @pl.when(j == HJ - 1)
    def _():
        ya_ref[...] = (ya_ref[...] + b2_ref[0]) * gs_ref[...]

        def scatter_row(i, _):
            tok = order_ref[e * CAP + i]
            o_ref[pl.ds(tok, 1), :] = ya_ref[pl.ds(i, 1), :]
            return 0
        lax.fori_loop(0, CAP, scatter_row, 0)


@jax.jit
def _k3(features, gp, order, W1r, b1, W2r, b2):
    return pl.pallas_call(
        _k3_body,
        grid=(E, HJ),
        in_specs=[
            pl.BlockSpec((B, D), lambda e, j: (0, 0)),
            pl.BlockSpec((B, E), lambda e, j: (0, 0)),
            pl.BlockSpec(memory_space=pltpu.SMEM),
            pl.BlockSpec((D, HB), lambda e, j: (e, j)),
            pl.BlockSpec((1, 1, HB), lambda e, j: (e, 0, j)),
            pl.BlockSpec((HB, O), lambda e, j: (e * HJ + j, 0)),
            pl.BlockSpec((1, 1, O), lambda e, j: (e, 0, 0)),
        ],
        out_specs=pl.BlockSpec((B, O), lambda e, j: (0, 0)),
        out_shape=jax.ShapeDtypeStruct((B, O), jnp.float32),
        scratch_shapes=[
            pltpu.VMEM((CAP, D), jnp.float32),
            pltpu.VMEM((CAP, 1), jnp.float32),
            pltpu.VMEM((CAP, O), jnp.float32),
        ],
        compiler_params=pltpu.CompilerParams(
            dimension_semantics=("arbitrary", "arbitrary")),
    )(features, gp, order, W1r, b1, W2r, b2)


def kernel(x, Wb, bb, Wg, bg, W1, b1, W2, b2):
    features, scores, gp = _k1(x, Wb, bb, Wg, bg)
    sorted_idx = jnp.argsort(-scores.reshape(-1), stable=True).astype(jnp.int32)
    order, pos = _k2(sorted_idx)
    return _k3(features, gp, order, W1.reshape(E * D, H), b1.reshape(E, 1, H),
               W2.reshape(E * H, O), b2.reshape(E, 1, O))


# speedup vs baseline: 1.0166x; 1.0166x over previous
import functools

import jax
import jax.numpy as jnp
from jax import lax
from jax.experimental import pallas as pl
from jax.experimental.pallas import tpu as pltpu
from jax.experimental.pallas import tpu_sc as plsc

B = 4096
E = 8
D = 1024
H = 2048
O = 1024
CAP = B // E
BLK = 512


def _k1_body(x_ref, wb_ref, bb_ref, wg_ref, bg_ref, feat_ref, sc_ref, gp_ref):
    f = jnp.dot(x_ref[...], wb_ref[...], preferred_element_type=jnp.float32)
    f = jnp.maximum(f + bb_ref[...], 0.0)
    feat_ref[...] = f
    s = jnp.dot(f, wg_ref[...], preferred_element_type=jnp.float32) + bg_ref[...]
    sc_ref[...] = s
    m = jnp.max(s, axis=1, keepdims=True)
    ex = jnp.exp(s - m)
    gp_ref[...] = ex / jnp.sum(ex, axis=1, keepdims=True)


@jax.jit
def _k1(x, Wb, bb, Wg, bg):
    return pl.pallas_call(
        _k1_body,
        grid=(B // BLK,),
        in_specs=[
            pl.BlockSpec((BLK, D), lambda i: (i, 0)),
            pl.BlockSpec((D, D), lambda i: (0, 0)),
            pl.BlockSpec((1, D), lambda i: (0, 0)),
            pl.BlockSpec((D, E), lambda i: (0, 0)),
            pl.BlockSpec((1, E), lambda i: (0, 0)),
        ],
        out_specs=[
            pl.BlockSpec((BLK, D), lambda i: (i, 0)),
            pl.BlockSpec((BLK, E), lambda i: (i, 0)),
            pl.BlockSpec((BLK, E), lambda i: (i, 0)),
        ],
        out_shape=[
            jax.ShapeDtypeStruct((B, D), jnp.float32),
            jax.ShapeDtypeStruct((B, E), jnp.float32),
            jax.ShapeDtypeStruct((B, E), jnp.float32),
        ],
        compiler_params=pltpu.CompilerParams(
            dimension_semantics=("arbitrary",)),
    )(x, Wb, bb.reshape(1, D), Wg, bg.reshape(1, E))


def _k2_body(sorted_ref, order_ref, pos_ref, caps_ref, nfull_ref,
             assigned_ref):
    for e in range(E):
        caps_ref[e] = CAP
    nfull_ref[0] = 0
    nfull_ref[1] = 0

    def init_b(b, _):
        assigned_ref[b] = -1
        return 0
    lax.fori_loop(0, B, init_b, 0, unroll=8)

    def chunk(c, _):
        @pl.when(nfull_ref[0] < E - 1)
        def _():
            def step(i, _):
                idx = sorted_ref[c * 1024 + i]
                b = lax.shift_right_logical(idx, 3)
                e = lax.bitwise_and(idx, 7)
                cap = caps_ref[e]
                a = assigned_ref[b]
                take = jnp.logical_and(a < 0, cap > 0)
                ti = take.astype(jnp.int32)
                assigned_ref[b] = jnp.where(take, e, a)
                caps_ref[e] = cap - ti
                filled = ti * (cap == 1).astype(jnp.int32)
                nfull_ref[0] = nfull_ref[0] + filled
                nfull_ref[1] = nfull_ref[1] + filled * e
                return 0
            lax.fori_loop(0, 1024, step, 0)
        return 0
    lax.fori_loop(0, (B * E) // 1024, chunk, 0)

    e_last = (E * (E - 1)) // 2 - nfull_ref[1]

    for e in range(E):
        caps_ref[e] = 0

    def place(b, _):
        a = assigned_ref[b]
        e = jnp.where(a < 0, e_last, a)
        k = caps_ref[e]
        p = e * CAP + k
        order_ref[p] = b
        pos_ref[b] = p
        caps_ref[e] = k + 1
        return 0
    lax.fori_loop(0, B, place, 0)


@jax.jit
def _k2(sorted_idx):
    return pl.pallas_call(
        _k2_body,
        in_specs=[pl.BlockSpec(memory_space=pltpu.SMEM)],
        out_specs=[pl.BlockSpec(memory_space=pltpu.SMEM),
                   pl.BlockSpec(memory_space=pltpu.SMEM)],
        out_shape=[jax.ShapeDtypeStruct((B,), jnp.int32),
                   jax.ShapeDtypeStruct((B,), jnp.int32)],
        scratch_shapes=[
            pltpu.SMEM((E,), jnp.int32),
            pltpu.SMEM((2,), jnp.int32),
            pltpu.SMEM((B,), jnp.int32),
        ],
    )(sorted_idx)


NW = 32
RPW = B // NW
RCH = RPW // 2


def _sc_gather_body(table_hbm, idx_hbm, out_hbm, idx_v, rows_v, sem):
    wid = lax.axis_index("s") * 2 + lax.axis_index("c")
    base = wid * RPW
    for half in range(2):
        lo = base + half * RCH
        pltpu.sync_copy(idx_hbm.at[pl.ds(lo, RCH)], idx_v)
        pltpu.async_copy(table_hbm.at[idx_v], rows_v, sem).wait()
        pltpu.sync_copy(rows_v, out_hbm.at[pl.ds(lo, RCH)])


@jax.jit
def _sc_gather(table, idx):
    k = functools.partial(
        pl.kernel,
        mesh=plsc.VectorSubcoreMesh(core_axis_name="c", subcore_axis_name="s"),
        out_type=jax.ShapeDtypeStruct((B, D), jnp.float32),
        scratch_types=[
            pltpu.VMEM((RCH,), jnp.int32),
            pltpu.VMEM((RCH, D), jnp.float32),
            pltpu.SemaphoreType.DMA,
        ],
    )(_sc_gather_body)
    return k(table, idx)


def _k3_body(xs_ref, gp_ref, order_ref, w1_ref, b1_ref, w2_ref, b2_ref,
             y_ref, gs_ref):
    e = pl.program_id(0)
    lane = lax.broadcasted_iota(jnp.int32, (1, E), 1)

    def gather_g(i, _):
        tok = order_ref[e * CAP + i]
        row = gp_ref[pl.ds(tok, 1), :]
        gs_ref[pl.ds(i, 1), :] = jnp.sum(
            jnp.where(lane == e, row, 0.0), axis=1, keepdims=True)
        return 0
    lax.fori_loop(0, CAP, gather_g, 0)

    h = jnp.dot(xs_ref[...], w1_ref[...], preferred_element_type=jnp.float32)
    h = jnp.maximum(h + b1_ref[0], 0.0)
    y = jnp.dot(h, w2_ref[...], preferred_element_type=jnp.float32)
    y_ref[...] = (y + b2_ref[0]) * gs_ref[...]


@jax.jit
def _k3(xs, gp, order, W1r, b1, W2r, b2):
    return pl.pallas_call(
        _k3_body,
        grid=(E,),
        in_specs=[
            pl.BlockSpec((CAP, D), lambda e: (e, 0)),
            pl.BlockSpec((B, E), lambda e: (0, 0)),
            pl.BlockSpec(memory_space=pltpu.SMEM),
            pl.BlockSpec((D, H), lambda e: (e, 0)),
            pl.BlockSpec((1, 1, H), lambda e: (e, 0, 0)),
            pl.BlockSpec((H, O), lambda e: (e, 0)),
            pl.BlockSpec((1, 1, O), lambda e: (e, 0, 0)),
        ],
        out_specs=pl.BlockSpec((CAP, O), lambda e: (e, 0)),
        out_shape=jax.ShapeDtypeStruct((B, O), jnp.float32),
        scratch_shapes=[
            pltpu.VMEM((CAP, 1), jnp.float32),
        ],
        compiler_params=pltpu.CompilerParams(
            dimension_semantics=("arbitrary",)),
    )(xs, gp, order, W1r, b1, W2r, b2)


def kernel(x, Wb, bb, Wg, bg, W1, b1, W2, b2):
    features, scores, gp = _k1(x, Wb, bb, Wg, bg)
    sorted_idx = jnp.argsort(-scores.reshape(-1), stable=True).astype(jnp.int32)
    order, pos = _k2(sorted_idx)
    xs = _sc_gather(features, order)
    y = _k3(xs, gp, order, W1.reshape(E * D, H), b1.reshape(E, 1, H),
            W2.reshape(E * H, O), b2.reshape(E, 1, O))
    return _sc_gather(y, pos)


# speedup vs baseline: 1.2448x; 1.2244x over previous
import functools

import jax
import jax.numpy as jnp
from jax import lax
from jax.experimental import pallas as pl
from jax.experimental.pallas import tpu as pltpu
from jax.experimental.pallas import tpu_sc as plsc

B = 4096
E = 8
D = 1024
H = 2048
O = 1024
CAP = B // E
BLK = 512


def _k1_body(x_ref, wb_ref, bb_ref, wg_ref, bg_ref, feat_ref, sc_ref, gp_ref):
    f = jnp.dot(x_ref[...], wb_ref[...], preferred_element_type=jnp.float32)
    f = jnp.maximum(f + bb_ref[...], 0.0)
    feat_ref[...] = f
    s = jnp.dot(f, wg_ref[...], preferred_element_type=jnp.float32) + bg_ref[...]
    sc_ref[...] = s
    m = jnp.max(s, axis=1, keepdims=True)
    ex = jnp.exp(s - m)
    gp_ref[...] = ex / jnp.sum(ex, axis=1, keepdims=True)


@jax.jit
def _k1(x, Wb, bb, Wg, bg):
    return pl.pallas_call(
        _k1_body,
        grid=(B // BLK,),
        in_specs=[
            pl.BlockSpec((BLK, D), lambda i: (i, 0)),
            pl.BlockSpec((D, D), lambda i: (0, 0)),
            pl.BlockSpec((1, D), lambda i: (0, 0)),
            pl.BlockSpec((D, E), lambda i: (0, 0)),
            pl.BlockSpec((1, E), lambda i: (0, 0)),
        ],
        out_specs=[
            pl.BlockSpec((BLK, D), lambda i: (i, 0)),
            pl.BlockSpec((BLK, E), lambda i: (i, 0)),
            pl.BlockSpec((BLK, E), lambda i: (i, 0)),
        ],
        out_shape=[
            jax.ShapeDtypeStruct((B, D), jnp.float32),
            jax.ShapeDtypeStruct((B, E), jnp.float32),
            jax.ShapeDtypeStruct((B, E), jnp.float32),
        ],
        compiler_params=pltpu.CompilerParams(
            dimension_semantics=("arbitrary",)),
    )(x, Wb, bb.reshape(1, D), Wg, bg.reshape(1, E))


def _k2_body(sorted_ref, order_ref, pos_ref, caps_ref, nfull_ref,
             assigned_ref):
    for e in range(E):
        caps_ref[e] = CAP
    nfull_ref[0] = 0
    nfull_ref[1] = 0

    def init_b(b, _):
        assigned_ref[b] = -1
        return 0
    lax.fori_loop(0, B, init_b, 0, unroll=8)

    def chunk(c, _):
        @pl.when(nfull_ref[0] < E - 1)
        def _():
            def step(i, _):
                idx = sorted_ref[c * 1024 + i]
                b = lax.shift_right_logical(idx, 3)
                e = lax.bitwise_and(idx, 7)
                cap = caps_ref[e]
                take = jnp.logical_and(assigned_ref[b] < 0, cap > 0)

                @pl.when(take)
                def _():
                    assigned_ref[b] = e
                    caps_ref[e] = cap - 1
                    filled = (cap == 1).astype(jnp.int32)
                    nfull_ref[0] = nfull_ref[0] + filled
                    nfull_ref[1] = nfull_ref[1] + filled * e
                return 0
            lax.fori_loop(0, 1024, step, 0, unroll=4)
        return 0
    lax.fori_loop(0, (B * E) // 1024, chunk, 0)

    e_last = (E * (E - 1)) // 2 - nfull_ref[1]

    for e in range(E):
        caps_ref[e] = 0

    def place(b, _):
        a = assigned_ref[b]
        e = jnp.where(a < 0, e_last, a)
        k = caps_ref[e]
        p = e * CAP + k
        order_ref[p] = b
        pos_ref[b] = p
        caps_ref[e] = k + 1
        return 0
    lax.fori_loop(0, B, place, 0)


@jax.jit
def _k2(sorted_idx):
    return pl.pallas_call(
        _k2_body,
        in_specs=[pl.BlockSpec(memory_space=pltpu.SMEM)],
        out_specs=[pl.BlockSpec(memory_space=pltpu.SMEM),
                   pl.BlockSpec(memory_space=pltpu.SMEM)],
        out_shape=[jax.ShapeDtypeStruct((B,), jnp.int32),
                   jax.ShapeDtypeStruct((B,), jnp.int32)],
        scratch_shapes=[
            pltpu.SMEM((E,), jnp.int32),
            pltpu.SMEM((2,), jnp.int32),
            pltpu.SMEM((B,), jnp.int32),
        ],
    )(sorted_idx)


NW = 32
RPW = B // NW
RCH = RPW // 2


def _sc_gather_body(table_hbm, idx_hbm, out_hbm, idx_v, rows_v, sem):
    wid = lax.axis_index("s") * 2 + lax.axis_index("c")
    base = wid * RPW
    for half in range(2):
        lo = base + half * RCH
        pltpu.sync_copy(idx_hbm.at[pl.ds(lo, RCH)], idx_v)
        pltpu.async_copy(table_hbm.at[idx_v], rows_v, sem).wait()
        pltpu.sync_copy(rows_v, out_hbm.at[pl.ds(lo, RCH)])


@jax.jit
def _sc_gather(table, idx):
    k = functools.partial(
        pl.kernel,
        mesh=plsc.VectorSubcoreMesh(core_axis_name="c", subcore_axis_name="s"),
        out_type=jax.ShapeDtypeStruct((B, D), jnp.float32),
        scratch_types=[
            pltpu.VMEM((RCH,), jnp.int32),
            pltpu.VMEM((RCH, D), jnp.float32),
            pltpu.SemaphoreType.DMA,
        ],
    )(_sc_gather_body)
    return k(table, idx)


def _k3_body(xs_ref, gp_ref, order_ref, w1_ref, b1_ref, w2_ref, b2_ref,
             y_ref, gs_ref):
    e = pl.program_id(0)
    lane = lax.broadcasted_iota(jnp.int32, (1, E), 1)

    def gather_g(i, _):
        tok = order_ref[e * CAP + i]
        row = gp_ref[pl.ds(tok, 1), :]
        gs_ref[pl.ds(i, 1), :] = jnp.sum(
            jnp.where(lane == e, row, 0.0), axis=1, keepdims=True)
        return 0
    lax.fori_loop(0, CAP, gather_g, 0)

    h = jnp.dot(xs_ref[...], w1_ref[...], preferred_element_type=jnp.float32)
    h = jnp.maximum(h + b1_ref[0], 0.0)
    y = jnp.dot(h, w2_ref[...], preferred_element_type=jnp.float32)
    y_ref[...] = (y + b2_ref[0]) * gs_ref[...]


@jax.jit
def _k3(xs, gp, order, W1r, b1, W2r, b2):
    return pl.pallas_call(
        _k3_body,
        grid=(E,),
        in_specs=[
            pl.BlockSpec((CAP, D), lambda e: (e, 0)),
            pl.BlockSpec((B, E), lambda e: (0, 0)),
            pl.BlockSpec(memory_space=pltpu.SMEM),
            pl.BlockSpec((D, H), lambda e: (e, 0)),
            pl.BlockSpec((1, 1, H), lambda e: (e, 0, 0)),
            pl.BlockSpec((H, O), lambda e: (e, 0)),
            pl.BlockSpec((1, 1, O), lambda e: (e, 0, 0)),
        ],
        out_specs=pl.BlockSpec((CAP, O), lambda e: (e, 0)),
        out_shape=jax.ShapeDtypeStruct((B, O), jnp.float32),
        scratch_shapes=[
            pltpu.VMEM((CAP, 1), jnp.float32),
        ],
        compiler_params=pltpu.CompilerParams(
            dimension_semantics=("arbitrary",)),
    )(xs, gp, order, W1r, b1, W2r, b2)


def kernel(x, Wb, bb, Wg, bg, W1, b1, W2, b2):
    features, scores, gp = _k1(x, Wb, bb, Wg, bg)
    sorted_idx = jnp.argsort(-scores.reshape(-1), stable=True).astype(jnp.int32)
    order, pos = _k2(sorted_idx)
    xs = _sc_gather(features, order)
    y = _k3(xs, gp, order, W1.reshape(E * D, H), b1.reshape(E, 1, H),
            W2.reshape(E * H, O), b2.reshape(E, 1, O))
    return _sc_gather(y, pos)
